# baseline (speedup 1.0000x reference)
import jax
import jax.numpy as jnp
from jax.experimental import pallas as pl
from jax.experimental.pallas import tpu as pltpu

_B, _S, _D, _H = 1, 8192, 1024, 16
_G = 16
_NG = _S // _G
_HD = _D // _H
_STRIDE = 2
_ALPHA = 0.5
_MOM = 0.9

_HPB = 2
_L = _HPB * _HD
_NBLK = _H // _HPB


def _roll_groups(v, sign):
    parts = []
    for j in range(_G):
        s = (sign * _STRIDE * j) % _NG
        sl = v[:, j:j + 1, :]
        if s:
            sl = jnp.concatenate([sl[s:], sl[:s]], axis=0)
        parts.append(sl)
    return jnp.concatenate(parts, axis=1)


def _ncn_block(x_ref, xa_ref, wmat_ref, exp_ref, oi_ref, oa_ref):
    y = x_ref[...].reshape(_NG, _G, _L)
    yav = xa_ref[...].reshape(_NG, _G, _L)
    z = _roll_groups(y, 1)
    u = (1.0 - _MOM) * _roll_groups(yav, 1)
    wmat = wmat_ref[0]
    expm = exp_ref[0]
    s2 = (1.0 - _MOM) ** 2
    nchunk = 2
    half = _NG // nchunk
    nh = half * _G

    state = [(z[c * half:(c + 1) * half], u[c * half:(c + 1) * half])
             for c in range(nchunk)]

    def dots(zh, sj):
        Pb = jax.lax.dot_general(
            zh.reshape(nh, _L).astype(jnp.bfloat16), wmat,
            (((1,), (0,)), ((), ())),
            preferred_element_type=jnp.float32)
        P = Pb[:, _L:_L + 4].reshape(half, _G, 4)
        zterm = Pb[:, 0:_L].reshape(half, _G, _L)
        sim = (_MOM ** (sj - 1)) * (P[:, :, 0:2] + P[:, sj:sj + 1, 2:4])
        sfb = jax.lax.dot_general(
            sim.reshape(nh, 2).astype(jnp.bfloat16), expm,
            (((1,), (0,)), ((), ())),
            preferred_element_type=jnp.float32).reshape(half, _G, _L)
        return zterm, sfb

    def update(zh, uh, zterm, sfb, sj):
        xj = zh[:, sj:sj + 1, :]
        m1 = zterm + sfb * xj
        uh = uh + jnp.maximum(m1, 0.01 * m1)
        return zh + uh, uh

    for sj in range(_G):
        zsfbs = [dots(zh, sj) for zh, _ in state]
        state = [update(zh, uh, zterm, sfb, sj)
                 for (zh, uh), (zterm, sfb) in zip(state, zsfbs)]

    z = jnp.concatenate([s[0] for s in state], axis=0)
    u = jnp.concatenate([s[1] for s in state], axis=0)
    mg = _MOM ** _G
    oi_ref[...] = _roll_groups(mg * z, -1).reshape(_S, _L)
    oa_ref[...] = _roll_groups((mg / (1.0 - _MOM)) * u, -1).reshape(_S, _L)


def kernel(x, xa, W):
    x2 = x.reshape(_S, _D)
    xa2 = xa.reshape(_S, _D)
    wi = W[:_D].reshape(_NBLK, _L)
    wj = W[_D:].reshape(_NBLK, _L)
    lane = jnp.arange(_L)
    m0 = (lane < _HD).astype(jnp.float32)
    m1 = 1.0 - m0
    dots4 = jnp.stack([wi * m0, wi * m1, wj * m0, wj * m1], axis=-1)
    zcoef = (((1.0 - _MOM) ** 2) * _ALPHA / _MOM)
    eye = jnp.broadcast_to(zcoef * jnp.eye(_L, dtype=jnp.float32),
                           (_NBLK, _L, _L))
    wmat = jnp.concatenate([eye, dots4], axis=-1).astype(jnp.bfloat16)
    scale = ((1.0 - _MOM) ** 2) * (1.0 - _ALPHA)
    expm = (scale * jnp.stack([m0, m1], axis=0))[None].astype(jnp.bfloat16)

    oi, oa = pl.pallas_call(
        _ncn_block,
        grid=(_NBLK,),
        in_specs=[
            pl.BlockSpec((_S, _L), lambda i: (0, i)),
            pl.BlockSpec((_S, _L), lambda i: (0, i)),
            pl.BlockSpec((1, _L, _L + 4), lambda i: (i, 0, 0)),
            pl.BlockSpec((1, 2, _L), lambda i: (0, 0, 0)),
        ],
        out_specs=[
            pl.BlockSpec((_S, _L), lambda i: (0, i)),
            pl.BlockSpec((_S, _L), lambda i: (0, i)),
        ],
        out_shape=[jax.ShapeDtypeStruct((_S, _D), jnp.float32)] * 2,
        compiler_params=pltpu.CompilerParams(
            dimension_semantics=("parallel",)),
    )(x2, xa2, wmat, expm)
    return oi.reshape(_B, _S, _D), oa.reshape(_B, _S, _D)


# speedup vs baseline: 1.7314x; 1.7314x over previous
import jax
import jax.numpy as jnp
from jax.experimental import pallas as pl
from jax.experimental.pallas import tpu as pltpu

_B, _S, _D, _H = 1, 8192, 1024, 16
_G = 16
_NG = _S // _G
_HD = _D // _H
_STRIDE = 2
_ALPHA = 0.5
_MOM = 0.9

_HPB = 2
_L = _HPB * _HD
_NBLK = _H // _HPB


def _roll_groups(v, sign):
    parts = []
    for j in range(_G):
        s = (sign * _STRIDE * j) % _NG
        sl = v[:, j:j + 1, :]
        if s:
            sl = jnp.concatenate([sl[s:], sl[:s]], axis=0)
        parts.append(sl)
    return jnp.concatenate(parts, axis=1)


def _ncn_block(x_ref, xa_ref, wmat_ref, exp_ref, oi_ref, oa_ref):
    y = x_ref[...].reshape(_NG, _G, _L)
    yav = xa_ref[...].reshape(_NG, _G, _L)
    z = _roll_groups(y, 1)
    u = (1.0 - _MOM) * _roll_groups(yav, 1)
    wmat = wmat_ref[0]
    expm = exp_ref[0]
    s2 = (1.0 - _MOM) ** 2
    nchunk = 2
    half = _NG // nchunk
    nh = half * _G

    state = [(z[c * half:(c + 1) * half], u[c * half:(c + 1) * half])
             for c in range(nchunk)]

    def dots(zh, sj):
        P = jax.lax.dot_general(
            zh.reshape(nh, _L).astype(jnp.bfloat16), wmat,
            (((1,), (0,)), ((), ())),
            preferred_element_type=jnp.float32).reshape(half, _G, 4)
        sim = (_MOM ** (sj - 1)) * (P[:, :, 0:2] + P[:, sj:sj + 1, 2:4])
        return jax.lax.dot_general(
            sim.reshape(nh, 2).astype(jnp.bfloat16), expm,
            (((1,), (0,)), ((), ())),
            preferred_element_type=jnp.float32).reshape(half, _G, _L)

    def update(zh, uh, sfb, sj):
        xj = zh[:, sj:sj + 1, :]
        m1 = (s2 * _ALPHA / _MOM) * zh + sfb * xj
        uh = uh + jnp.maximum(m1, 0.01 * m1)
        return zh + uh, uh

    for sj in range(_G):
        sfbs = [dots(zh, sj) for zh, _ in state]
        state = [update(zh, uh, sfb, sj)
                 for (zh, uh), sfb in zip(state, sfbs)]

    z = jnp.concatenate([s[0] for s in state], axis=0)
    u = jnp.concatenate([s[1] for s in state], axis=0)
    mg = _MOM ** _G
    oi_ref[...] = _roll_groups(mg * z, -1).reshape(_S, _L)
    oa_ref[...] = _roll_groups((mg / (1.0 - _MOM)) * u, -1).reshape(_S, _L)


def kernel(x, xa, W):
    x2 = x.reshape(_S, _D)
    xa2 = xa.reshape(_S, _D)
    wi = W[:_D].reshape(_NBLK, _L)
    wj = W[_D:].reshape(_NBLK, _L)
    lane = jnp.arange(_L)
    m0 = (lane < _HD).astype(jnp.float32)
    m1 = 1.0 - m0
    wmat = jnp.stack([wi * m0, wi * m1, wj * m0, wj * m1],
                     axis=-1).astype(jnp.bfloat16)
    scale = ((1.0 - _MOM) ** 2) * (1.0 - _ALPHA)
    expm = (scale * jnp.stack([m0, m1], axis=0))[None].astype(jnp.bfloat16)

    oi, oa = pl.pallas_call(
        _ncn_block,
        grid=(_NBLK,),
        in_specs=[
            pl.BlockSpec((_S, _L), lambda i: (0, i)),
            pl.BlockSpec((_S, _L), lambda i: (0, i)),
            pl.BlockSpec((1, _L, 4), lambda i: (i, 0, 0)),
            pl.BlockSpec((1, 2, _L), lambda i: (0, 0, 0)),
        ],
        out_specs=[
            pl.BlockSpec((_S, _L), lambda i: (0, i)),
            pl.BlockSpec((_S, _L), lambda i: (0, i)),
        ],
        out_shape=[jax.ShapeDtypeStruct((_S, _D), jnp.float32)] * 2,
        compiler_params=pltpu.CompilerParams(
            dimension_semantics=("parallel",)),
    )(x2, xa2, wmat, expm)
    return oi.reshape(_B, _S, _D), oa.reshape(_B, _S, _D)
